# baseline (speedup 1.0000x reference)
import functools

import jax
import jax.numpy as jnp
from jax import lax
from jax.experimental import pallas as pl
from jax.experimental.pallas import tpu as pltpu
from jax.experimental.pallas import tpu_sc as plsc

_B = 16384
_V = 8192
_D = 1024
_NC = 2
_NS = 16
_NW = _NC * _NS
_BPW = _B // _NW
_C = 32
_NBUF = 3
_NCHUNK = _BPW // _C


def _sc_gather(table, t):
    mesh = plsc.VectorSubcoreMesh(
        core_axis_name="c", subcore_axis_name="s",
        num_cores=_NC, num_subcores=_NS,
    )

    @functools.partial(
        pl.kernel,
        out_type=jax.ShapeDtypeStruct((_B, _D), jnp.float32),
        mesh=mesh,
        scratch_types=[
            pltpu.VMEM((_BPW,), jnp.int32),
            pltpu.VMEM((_NBUF, _C, _D), jnp.float32),
            pltpu.SemaphoreType.DMA,
            pltpu.SemaphoreType.DMA,
        ],
    )
    def body(table_hbm, t_hbm, out_hbm, idx_v, rows_v, sem_r, sem_w):
        wid = lax.axis_index("s") * _NC + lax.axis_index("c")
        base = wid * _BPW
        pltpu.sync_copy(t_hbm.at[pl.ds(base, _BPW)], idx_v)

        def gather(g, buf):
            return pltpu.make_async_copy(
                table_hbm.at[idx_v.at[pl.ds(g * _C, _C)]],
                rows_v.at[buf],
                sem_r,
            )

        def write(g, buf):
            return pltpu.make_async_copy(
                rows_v.at[buf],
                out_hbm.at[pl.ds(base + g * _C, _C)],
                sem_w,
            )

        for g in range(_NBUF - 1):
            gather(g, g % _NBUF).start()
        for g in range(_NCHUNK):
            buf = g % _NBUF
            nxt = g + _NBUF - 1
            if nxt < _NCHUNK:
                if g >= 1:
                    write(g - 1, (g - 1) % _NBUF).wait()
                gather(nxt, nxt % _NBUF).start()
            gather(g, buf).wait()
            write(g, buf).start()
        for g in range(_NCHUNK - _NBUF, _NCHUNK):
            write(g, g % _NBUF).wait()

    return body(table, t)


def kernel(t, pos_embeddings):
    return _sc_gather(pos_embeddings, t.astype(jnp.int32))


# speedup vs baseline: 1.0181x; 1.0181x over previous
import functools

import jax
import jax.numpy as jnp
from jax import lax
from jax.experimental import pallas as pl
from jax.experimental.pallas import tpu as pltpu
from jax.experimental.pallas import tpu_sc as plsc

_B = 16384
_V = 8192
_D = 1024
_NC = 2
_NS = 16
_NW = _NC * _NS
_BPW = _B // _NW
_C = 56
_NBUF = 2
_CHUNKS = []
_off = 0
while _off < _BPW:
    _CHUNKS.append((_off, min(_C, _BPW - _off)))
    _off += _C
_NCHUNK = len(_CHUNKS)


def _sc_gather(table, t):
    mesh = plsc.VectorSubcoreMesh(
        core_axis_name="c", subcore_axis_name="s",
        num_cores=_NC, num_subcores=_NS,
    )

    @functools.partial(
        pl.kernel,
        out_type=jax.ShapeDtypeStruct((_B, _D), jnp.float32),
        mesh=mesh,
        scratch_types=[
            pltpu.VMEM((_BPW,), jnp.int32),
            pltpu.VMEM((_NBUF, _C, _D), jnp.float32),
            pltpu.SemaphoreType.DMA,
            pltpu.SemaphoreType.DMA,
        ],
    )
    def body(table_hbm, t_hbm, out_hbm, idx_v, rows_v, sem_r, sem_w):
        wid = lax.axis_index("s") * _NC + lax.axis_index("c")
        base = wid * _BPW
        pltpu.sync_copy(t_hbm.at[pl.ds(base, _BPW)], idx_v)

        def gather(g, buf):
            off, c = _CHUNKS[g]
            return pltpu.make_async_copy(
                table_hbm.at[idx_v.at[pl.ds(off, c)]],
                rows_v.at[buf, pl.ds(0, c)],
                sem_r,
            )

        def write(g, buf):
            off, c = _CHUNKS[g]
            return pltpu.make_async_copy(
                rows_v.at[buf, pl.ds(0, c)],
                out_hbm.at[pl.ds(base + off, c)],
                sem_w,
            )

        for g in range(_NBUF - 1):
            gather(g, g % _NBUF).start()
        for g in range(_NCHUNK):
            buf = g % _NBUF
            nxt = g + _NBUF - 1
            if nxt < _NCHUNK:
                if g >= 1:
                    write(g - 1, (g - 1) % _NBUF).wait()
                gather(nxt, nxt % _NBUF).start()
            gather(g, buf).wait()
            write(g, buf).start()
        for g in range(_NCHUNK - _NBUF, _NCHUNK):
            write(g, g % _NBUF).wait()

    return body(table, t)


def kernel(t, pos_embeddings):
    return _sc_gather(pos_embeddings, t.astype(jnp.int32))
